# baseline (speedup 1.0000x reference)
import functools

import jax
import jax.numpy as jnp
from jax.experimental import pallas as pl
from jax.experimental.pallas import tpu as pltpu

BATCH = 4096
NSTEP = 11
LENGTH = 20
INPUT_SIZE = 64


def _tc_gather(weight, steps):
    def body(s_ref, w_ref, out_ref):
        out_ref[...] = w_ref[0]

    grid_spec = pltpu.PrefetchScalarGridSpec(
        num_scalar_prefetch=1,
        grid=(BATCH,),
        in_specs=[pl.BlockSpec((1, 1, LENGTH, INPUT_SIZE),
                               lambda b, s_ref: (b, s_ref[b], 0, 0))],
        out_specs=pl.BlockSpec((1, LENGTH, INPUT_SIZE),
                               lambda b, s_ref: (b, 0, 0)),
    )
    return pl.pallas_call(
        body,
        grid_spec=grid_spec,
        out_shape=jax.ShapeDtypeStruct((BATCH, LENGTH, INPUT_SIZE),
                                       jnp.float32),
    )(steps, weight)


def kernel(embeddings, conditions, steps, weight, lengths):
    out = _tc_gather(weight, steps.astype(jnp.int32))
    return (out, lengths)


# speedup vs baseline: 3.9246x; 3.9246x over previous
import functools

import jax
import jax.numpy as jnp
from jax.experimental import pallas as pl
from jax.experimental.pallas import tpu as pltpu

BATCH = 4096
NSTEP = 11
LENGTH = 20
INPUT_SIZE = 64

BLK = 64
NBLK = BATCH // BLK


def _tc_gather(weight, steps):
    def body(s_ref, w_ref, out_ref):
        i = pl.program_id(0)
        base = i * BLK
        for j in range(BLK):
            out_ref[j] = w_ref[j, s_ref[base + j]]

    grid_spec = pltpu.PrefetchScalarGridSpec(
        num_scalar_prefetch=1,
        grid=(NBLK,),
        in_specs=[pl.BlockSpec((BLK, NSTEP, LENGTH, INPUT_SIZE),
                               lambda i, s_ref: (i, 0, 0, 0))],
        out_specs=pl.BlockSpec((BLK, LENGTH, INPUT_SIZE),
                               lambda i, s_ref: (i, 0, 0)),
    )
    return pl.pallas_call(
        body,
        grid_spec=grid_spec,
        out_shape=jax.ShapeDtypeStruct((BATCH, LENGTH, INPUT_SIZE),
                                       jnp.float32),
    )(steps, weight)


def kernel(embeddings, conditions, steps, weight, lengths):
    out = _tc_gather(weight, steps.astype(jnp.int32))
    return (out, lengths)


# speedup vs baseline: 8.6848x; 2.2129x over previous
import functools

import jax
import jax.numpy as jnp
from jax import lax
from jax.experimental import pallas as pl
from jax.experimental.pallas import tpu as pltpu
from jax.experimental.pallas import tpu_sc as plsc

BATCH = 4096
NSTEP = 11
LENGTH = 20
INPUT_SIZE = 64

NC = 2
NS = 16
NW = NC * NS
B_PER_W = BATCH // NW
GROUP = 16
NGROUP = B_PER_W // GROUP


def _gather_rows(table, steps):
    mesh = plsc.VectorSubcoreMesh(core_axis_name="c", subcore_axis_name="s")

    @functools.partial(
        pl.kernel,
        mesh=mesh,
        out_type=jax.ShapeDtypeStruct((BATCH, LENGTH, INPUT_SIZE),
                                      jnp.float32),
        scratch_types=[
            pltpu.VMEM((B_PER_W,), jnp.int32),
            pltpu.VMEM((GROUP, LENGTH, INPUT_SIZE), jnp.float32),
            pltpu.SemaphoreType.DMA,
        ],
    )
    def k(table_hbm, steps_hbm, out_hbm, steps_v, rows_v, sem):
        wid = lax.axis_index("s") * NC + lax.axis_index("c")
        start = wid * B_PER_W
        pltpu.sync_copy(steps_hbm.at[pl.ds(start, B_PER_W)], steps_v)

        @pl.loop(0, NGROUP)
        def _(g):
            base = g * GROUP
            svec = steps_v[pl.ds(base, GROUP)]
            copies = []
            for j in range(GROUP):
                idx = (start + base + j) * NSTEP + svec[j]
                copies.append(
                    pltpu.make_async_copy(table_hbm.at[idx], rows_v.at[j],
                                          sem))
            for c in copies:
                c.start()
            for c in copies:
                c.wait()
            pltpu.sync_copy(rows_v,
                            out_hbm.at[pl.ds(start + base, GROUP)])

    return k(table, steps)


def kernel(embeddings, conditions, steps, weight, lengths):
    table = weight.reshape(BATCH * NSTEP, LENGTH, INPUT_SIZE)
    out = _gather_rows(table, steps.astype(jnp.int32))
    return (out, lengths)
